# baseline (speedup 1.0000x reference)
import functools

import jax
import jax.numpy as jnp
from jax import lax
from jax.experimental import pallas as pl
from jax.experimental.pallas import tpu as pltpu
from jax.experimental.pallas import tpu_sc as plsc

N = 10000
E = 160000
D = 256
H = 128

NC = 2
NT = 16
NW = NC * NT
NP = 10240
EP = 163840
EPW = EP // NW
CHUNK = 128
NCH = EPW // CHUNK
C0CH = 24
C1CH = 80 - C0CH
RPT = NP // NT

_mesh = plsc.VectorSubcoreMesh(core_axis_name="c", subcore_axis_name="s")


@functools.partial(
    pl.kernel,
    out_type=jax.ShapeDtypeStruct((NW, NP), jnp.float32),
    mesh=_mesh,
    scratch_types=[
        pltpu.VMEM((NCH, CHUNK), jnp.int32),
        pltpu.VMEM((NP,), jnp.float32),
    ],
    compiler_params=pltpu.CompilerParams(needs_layout_passes=False),
)
def _sc_deg(dstp2, out, idx_d, deg):
    c = lax.axis_index("c")
    s = lax.axis_index("s")
    wid = c * NT + s

    def zbody(i, carry):
        deg[pl.ds(i * 16, 16)] = jnp.zeros((16,), jnp.float32)
        return carry

    lax.fori_loop(0, NP // 16, zbody, 0)
    pltpu.sync_copy(dstp2.at[pl.ds(wid * NCH, NCH)], idx_d)
    ones = jnp.ones((16,), jnp.float32)

    def body(j, carry):
        def inner(k, carry2):
            idx = idx_d[j, pl.ds(k * 16, 16)]
            plsc.addupdate_scatter(deg, [idx], ones)
            return carry2

        return lax.fori_loop(0, CHUNK // 16, inner, carry)

    lax.fori_loop(0, NCH, body, 0)
    pltpu.sync_copy(deg, out.at[wid])


@functools.partial(
    pl.kernel,
    out_type=jax.ShapeDtypeStruct((NC, NP, H), jnp.float32),
    mesh=_mesh,
    scratch_types=[
        pltpu.VMEM((C1CH, CHUNK), jnp.int32),
        pltpu.VMEM((C1CH, CHUNK), jnp.int32),
        pltpu.VMEM((CHUNK, H), jnp.float32),
        pltpu.VMEM_SHARED((NP, H), jnp.float32),
        pltpu.SemaphoreType.DMA,
    ],
)
def _sc_pass(table, srcp2, dstp2, zeros, out, idx_s, idx_d, rows0,
             acc, sem0):
    c = lax.axis_index("c")
    s = lax.axis_index("s")
    pltpu.sync_copy(zeros, acc.at[pl.ds(s * RPT, RPT)])
    base = lax.select(c == 0, s * C0CH, NT * C0CH + s * C1CH)
    nch_me = lax.select(c == 0, C0CH, C1CH)
    pltpu.sync_copy(srcp2.at[pl.ds(base, C1CH)], idx_s)
    pltpu.sync_copy(dstp2.at[pl.ds(base, C1CH)], idx_d)
    plsc.subcore_barrier()

    def body(j, carry):
        pltpu.async_copy(table.at[idx_s.at[j]], rows0, sem0).wait()
        pltpu.sync_copy(rows0, acc.at[idx_d.at[j]], add=True)
        return carry

    lax.fori_loop(0, nch_me, body, 0)
    plsc.subcore_barrier()
    pltpu.sync_copy(acc.at[pl.ds(s * RPT, RPT)],
                    out.at[c].at[pl.ds(s * RPT, RPT)])


_R = 1000


def _gelu(v):
    return 0.5 * v * (1.0 + lax.erf(v * 0.7071067811865476))


def _tc_a(x, W1, degp):
    def body(x_ref, w_ref, deg_ref, h1s_ref, dinvb_ref):
        dsum = jnp.sum(deg_ref[...], axis=1, keepdims=True)
        dinv = lax.rsqrt(dsum + 1.0)
        dinvb = jnp.broadcast_to(dinv, (_R, H))
        h = jnp.dot(x_ref[...], w_ref[...],
                    preferred_element_type=jnp.float32)
        h1s_ref[...] = h * dinvb
        dinvb_ref[...] = dinvb

    return pl.pallas_call(
        body,
        grid=(N // _R,),
        in_specs=[
            pl.BlockSpec((_R, D), lambda m: (m, 0)),
            pl.BlockSpec((D, H), lambda m: (0, 0)),
            pl.BlockSpec((_R, NW), lambda m: (m, 0)),
        ],
        out_specs=[
            pl.BlockSpec((_R, H), lambda m: (m, 0)),
            pl.BlockSpec((_R, H), lambda m: (m, 0)),
        ],
        out_shape=[
            jax.ShapeDtypeStruct((N, H), jnp.float32),
            jax.ShapeDtypeStruct((N, H), jnp.float32),
        ],
    )(x, W1, degp)


def _tc_b(s1, h1s, dinvb, b1, W2):
    def body(s_ref, h1s_ref, dinvb_ref, b1_ref, w2_ref, h2s_ref):
        dinvb = dinvb_ref[...]
        agg = dinvb * (s_ref[0] + s_ref[1] + h1s_ref[...]) + b1_ref[...]
        y = _gelu(agg)
        mu = jnp.mean(y, axis=-1, keepdims=True)
        var = jnp.mean((y - mu) ** 2, axis=-1, keepdims=True)
        ln = (y - mu) / jnp.sqrt(var + 1e-5)
        h2 = jnp.dot(ln, w2_ref[...], preferred_element_type=jnp.float32)
        h2s_ref[...] = h2 * dinvb

    return pl.pallas_call(
        body,
        grid=(N // _R,),
        in_specs=[
            pl.BlockSpec((NC, _R, H), lambda m: (0, m, 0)),
            pl.BlockSpec((_R, H), lambda m: (m, 0)),
            pl.BlockSpec((_R, H), lambda m: (m, 0)),
            pl.BlockSpec((1, H), lambda m: (0, 0)),
            pl.BlockSpec((H, H), lambda m: (0, 0)),
        ],
        out_specs=pl.BlockSpec((_R, H), lambda m: (m, 0)),
        out_shape=jax.ShapeDtypeStruct((N, H), jnp.float32),
    )(s1, h1s, dinvb, b1, W2)


def _tc_c(s2, h2s, dinvb, b2):
    def body(s_ref, h2s_ref, dinvb_ref, b2_ref, out_ref):
        agg = (dinvb_ref[...] * (s_ref[0] + s_ref[1] + h2s_ref[...])
               + b2_ref[...])
        out_ref[...] = _gelu(agg)

    return pl.pallas_call(
        body,
        grid=(N // _R,),
        in_specs=[
            pl.BlockSpec((NC, _R, H), lambda m: (0, m, 0)),
            pl.BlockSpec((_R, H), lambda m: (m, 0)),
            pl.BlockSpec((_R, H), lambda m: (m, 0)),
            pl.BlockSpec((1, H), lambda m: (0, 0)),
        ],
        out_specs=pl.BlockSpec((_R, H), lambda m: (m, 0)),
        out_shape=jax.ShapeDtypeStruct((N, H), jnp.float32),
    )(s2, h2s, dinvb, b2)


def kernel(x, edge_index, W1, b1, W2, b2):
    src = edge_index[0]
    dst = edge_index[1]
    pad = EP - E
    srcp = jnp.concatenate(
        [src, jnp.zeros((pad,), jnp.int32)]).reshape(EP // CHUNK, CHUNK)
    dstp = jnp.concatenate(
        [dst, jnp.full((pad,), N, jnp.int32)]).reshape(EP // CHUNK, CHUNK)
    zeros = jnp.zeros((RPT, H), jnp.float32)

    degp = _sc_deg(dstp)
    h1s, dinvb = _tc_a(x, W1, degp[:, :N].T)
    s1 = _sc_pass(h1s, srcp, dstp, zeros)
    h2s = _tc_b(s1[:, :N, :], h1s, dinvb, b1.reshape(1, H), W2)
    s2 = _sc_pass(h2s, srcp, dstp, zeros)
    return _tc_c(s2[:, :N, :], h2s, dinvb, b2.reshape(1, H))


# speedup vs baseline: 1.2108x; 1.2108x over previous
import functools

import jax
import jax.numpy as jnp
from jax import lax
from jax.experimental import pallas as pl
from jax.experimental.pallas import tpu as pltpu
from jax.experimental.pallas import tpu_sc as plsc

N = 10000
E = 160000
D = 256
H = 128

NC = 2
NT = 16
NW = NC * NT
NP = 10240
EP = 163840
EPW = EP // NW
CHUNK = 128
NCH = EPW // CHUNK
C0CH = 56
C1CH = 80 - C0CH
MXCH = max(C0CH, C1CH)
RPT = NP // NT

_mesh = plsc.VectorSubcoreMesh(core_axis_name="c", subcore_axis_name="s")


@functools.partial(
    pl.kernel,
    out_type=jax.ShapeDtypeStruct((NW, NP), jnp.float32),
    mesh=_mesh,
    scratch_types=[
        pltpu.VMEM((NCH, CHUNK), jnp.int32),
        pltpu.VMEM((NP,), jnp.float32),
    ],
    compiler_params=pltpu.CompilerParams(needs_layout_passes=False),
)
def _sc_deg(dstp2, out, idx_d, deg):
    c = lax.axis_index("c")
    s = lax.axis_index("s")
    wid = c * NT + s

    def zbody(i, carry):
        deg[pl.ds(i * 16, 16)] = jnp.zeros((16,), jnp.float32)
        return carry

    lax.fori_loop(0, NP // 16, zbody, 0)
    pltpu.sync_copy(dstp2.at[pl.ds(wid * NCH, NCH)], idx_d)
    ones = jnp.ones((16,), jnp.float32)

    def body(j, carry):
        def inner(k, carry2):
            idx = idx_d[j, pl.ds(k * 16, 16)]
            plsc.addupdate_scatter(deg, [idx], ones)
            return carry2

        return lax.fori_loop(0, CHUNK // 16, inner, carry)

    lax.fori_loop(0, NCH, body, 0)
    pltpu.sync_copy(deg, out.at[wid])


@functools.partial(
    pl.kernel,
    out_type=jax.ShapeDtypeStruct((NC, NP, H), jnp.float32),
    mesh=_mesh,
    scratch_types=[
        pltpu.VMEM((MXCH, CHUNK), jnp.int32),
        pltpu.VMEM((MXCH, CHUNK), jnp.int32),
        pltpu.VMEM((CHUNK, H), jnp.float32),
        pltpu.VMEM_SHARED((NP, H), jnp.float32),
        pltpu.SemaphoreType.DMA,
    ],
)
def _sc_pass(table, srcp2, dstp2, zeros, out, idx_s, idx_d, rows0,
             acc, sem0):
    c = lax.axis_index("c")
    s = lax.axis_index("s")
    pltpu.sync_copy(zeros, acc.at[pl.ds(s * RPT, RPT)])
    nch_me = lax.select(c == 0, C0CH, C1CH)

    @pl.when(c == 0)
    def _():
        pltpu.sync_copy(srcp2.at[pl.ds(s * C0CH, C0CH)],
                        idx_s.at[pl.ds(0, C0CH)])
        pltpu.sync_copy(dstp2.at[pl.ds(s * C0CH, C0CH)],
                        idx_d.at[pl.ds(0, C0CH)])

    @pl.when(c == 1)
    def _():
        pltpu.sync_copy(srcp2.at[pl.ds(NT * C0CH + s * C1CH, C1CH)],
                        idx_s.at[pl.ds(0, C1CH)])
        pltpu.sync_copy(dstp2.at[pl.ds(NT * C0CH + s * C1CH, C1CH)],
                        idx_d.at[pl.ds(0, C1CH)])

    plsc.subcore_barrier()

    def body(j, carry):
        pltpu.async_copy(table.at[idx_s.at[j]], rows0, sem0).wait()
        pltpu.sync_copy(rows0, acc.at[idx_d.at[j]], add=True)
        return carry

    lax.fori_loop(0, nch_me, body, 0)
    plsc.subcore_barrier()
    pltpu.sync_copy(acc.at[pl.ds(s * RPT, RPT)],
                    out.at[c].at[pl.ds(s * RPT, RPT)])


_R = 1000


def _gelu(v):
    return 0.5 * v * (1.0 + lax.erf(v * 0.7071067811865476))


def _tc_a(x, W1, degp):
    def body(x_ref, w_ref, deg_ref, h1s_ref, dinvb_ref):
        dsum = jnp.sum(deg_ref[...], axis=1, keepdims=True)
        dinv = lax.rsqrt(dsum + 1.0)
        dinvb = jnp.broadcast_to(dinv, (_R, H))
        h = jnp.dot(x_ref[...], w_ref[...],
                    preferred_element_type=jnp.float32)
        h1s_ref[...] = h * dinvb
        dinvb_ref[...] = dinvb

    return pl.pallas_call(
        body,
        grid=(N // _R,),
        in_specs=[
            pl.BlockSpec((_R, D), lambda m: (m, 0)),
            pl.BlockSpec((D, H), lambda m: (0, 0)),
            pl.BlockSpec((_R, NW), lambda m: (m, 0)),
        ],
        out_specs=[
            pl.BlockSpec((_R, H), lambda m: (m, 0)),
            pl.BlockSpec((_R, H), lambda m: (m, 0)),
        ],
        out_shape=[
            jax.ShapeDtypeStruct((N, H), jnp.float32),
            jax.ShapeDtypeStruct((N, H), jnp.float32),
        ],
    )(x, W1, degp)


def _tc_b(s1, h1s, dinvb, b1, W2):
    def body(s_ref, h1s_ref, dinvb_ref, b1_ref, w2_ref, h2s_ref):
        dinvb = dinvb_ref[...]
        agg = dinvb * (s_ref[0] + s_ref[1] + h1s_ref[...]) + b1_ref[...]
        y = _gelu(agg)
        mu = jnp.mean(y, axis=-1, keepdims=True)
        var = jnp.mean((y - mu) ** 2, axis=-1, keepdims=True)
        ln = (y - mu) / jnp.sqrt(var + 1e-5)
        h2 = jnp.dot(ln, w2_ref[...], preferred_element_type=jnp.float32)
        h2s_ref[...] = h2 * dinvb

    return pl.pallas_call(
        body,
        grid=(N // _R,),
        in_specs=[
            pl.BlockSpec((NC, _R, H), lambda m: (0, m, 0)),
            pl.BlockSpec((_R, H), lambda m: (m, 0)),
            pl.BlockSpec((_R, H), lambda m: (m, 0)),
            pl.BlockSpec((1, H), lambda m: (0, 0)),
            pl.BlockSpec((H, H), lambda m: (0, 0)),
        ],
        out_specs=pl.BlockSpec((_R, H), lambda m: (m, 0)),
        out_shape=jax.ShapeDtypeStruct((N, H), jnp.float32),
    )(s1, h1s, dinvb, b1, W2)


def _tc_c(s2, h2s, dinvb, b2):
    def body(s_ref, h2s_ref, dinvb_ref, b2_ref, out_ref):
        agg = (dinvb_ref[...] * (s_ref[0] + s_ref[1] + h2s_ref[...])
               + b2_ref[...])
        out_ref[...] = _gelu(agg)

    return pl.pallas_call(
        body,
        grid=(N // _R,),
        in_specs=[
            pl.BlockSpec((NC, _R, H), lambda m: (0, m, 0)),
            pl.BlockSpec((_R, H), lambda m: (m, 0)),
            pl.BlockSpec((_R, H), lambda m: (m, 0)),
            pl.BlockSpec((1, H), lambda m: (0, 0)),
        ],
        out_specs=pl.BlockSpec((_R, H), lambda m: (m, 0)),
        out_shape=jax.ShapeDtypeStruct((N, H), jnp.float32),
    )(s2, h2s, dinvb, b2)


def kernel(x, edge_index, W1, b1, W2, b2):
    src = edge_index[0]
    dst = edge_index[1]
    pad = EP - E
    srcp = jnp.concatenate(
        [src, jnp.zeros((pad,), jnp.int32)]).reshape(EP // CHUNK, CHUNK)
    dstp = jnp.concatenate(
        [dst, jnp.full((pad,), N, jnp.int32)]).reshape(EP // CHUNK, CHUNK)
    zeros = jnp.zeros((RPT, H), jnp.float32)

    degp = _sc_deg(dstp)
    h1s, dinvb = _tc_a(x, W1, degp[:, :N].T)
    s1 = _sc_pass(h1s, srcp, dstp, zeros)
    h2s = _tc_b(s1[:, :N, :], h1s, dinvb, b1.reshape(1, H), W2)
    s2 = _sc_pass(h2s, srcp, dstp, zeros)
    return _tc_c(s2[:, :N, :], h2s, dinvb, b2.reshape(1, H))


# speedup vs baseline: 1.2334x; 1.0187x over previous
import functools

import jax
import jax.numpy as jnp
from jax import lax
from jax.experimental import pallas as pl
from jax.experimental.pallas import tpu as pltpu
from jax.experimental.pallas import tpu_sc as plsc

N = 10000
E = 160000
D = 256
H = 128

NC = 2
NT = 16
NW = NC * NT
NP = 10240
EP = 163840
EPW = EP // NW
CHUNK = 128
NCH = EPW // CHUNK
C0CH = 56
C1CH = 80 - C0CH
MXCH = max(C0CH, C1CH)
RPT = NP // NT

_mesh = plsc.VectorSubcoreMesh(core_axis_name="c", subcore_axis_name="s")


@functools.partial(
    pl.kernel,
    out_type=jax.ShapeDtypeStruct((NW, NP), jnp.float32),
    mesh=_mesh,
    scratch_types=[
        pltpu.VMEM((NCH, CHUNK), jnp.int32),
        pltpu.VMEM((NP,), jnp.float32),
    ],
    compiler_params=pltpu.CompilerParams(needs_layout_passes=False),
)
def _sc_deg(dstp2, out, idx_d, deg):
    c = lax.axis_index("c")
    s = lax.axis_index("s")
    wid = c * NT + s

    def zbody(i, carry):
        deg[pl.ds(i * 16, 16)] = jnp.zeros((16,), jnp.float32)
        return carry

    lax.fori_loop(0, NP // 16, zbody, 0)
    pltpu.sync_copy(dstp2.at[pl.ds(wid * NCH, NCH)], idx_d)
    ones = jnp.ones((16,), jnp.float32)

    def body(j, carry):
        def inner(k, carry2):
            idx = idx_d[j, pl.ds(k * 16, 16)]
            plsc.addupdate_scatter(deg, [idx], ones)
            return carry2

        return lax.fori_loop(0, CHUNK // 16, inner, carry)

    lax.fori_loop(0, NCH, body, 0)
    pltpu.sync_copy(deg, out.at[wid])


@functools.partial(
    pl.kernel,
    out_type=jax.ShapeDtypeStruct((NC, NP, H), jnp.float32),
    mesh=_mesh,
    scratch_types=[
        pltpu.VMEM((MXCH, CHUNK), jnp.int32),
        pltpu.VMEM((MXCH, CHUNK), jnp.int32),
        pltpu.VMEM((CHUNK, H), jnp.float32),
        pltpu.VMEM((CHUNK, H), jnp.float32),
        pltpu.VMEM_SHARED((NP, H), jnp.float32),
        pltpu.SemaphoreType.DMA,
    ],
)
def _sc_pass(table, srcp2, dstp2, zeros, out, idx_s, idx_d, rows0, rows1,
             acc, sem0):
    c = lax.axis_index("c")
    s = lax.axis_index("s")
    pltpu.sync_copy(zeros, acc.at[pl.ds(s * RPT, RPT)])
    nch_me = lax.select(c == 0, C0CH, C1CH)

    @pl.when(c == 0)
    def _():
        pltpu.sync_copy(srcp2.at[pl.ds(s * C0CH, C0CH)],
                        idx_s.at[pl.ds(0, C0CH)])
        pltpu.sync_copy(dstp2.at[pl.ds(s * C0CH, C0CH)],
                        idx_d.at[pl.ds(0, C0CH)])

    @pl.when(c == 1)
    def _():
        pltpu.sync_copy(srcp2.at[pl.ds(NT * C0CH + s * C1CH, C1CH)],
                        idx_s.at[pl.ds(0, C1CH)])
        pltpu.sync_copy(dstp2.at[pl.ds(NT * C0CH + s * C1CH, C1CH)],
                        idx_d.at[pl.ds(0, C1CH)])

    plsc.subcore_barrier()

    def body(j2, carry):
        j = j2 * 2
        pltpu.async_copy(table.at[idx_s.at[j]], rows0, sem0)
        pltpu.async_copy(table.at[idx_s.at[j + 1]], rows1, sem0)
        pltpu.make_async_copy(table.at[idx_s.at[j]], rows0, sem0).wait()
        pltpu.sync_copy(rows0, acc.at[idx_d.at[j]], add=True)
        pltpu.make_async_copy(table.at[idx_s.at[j + 1]], rows1, sem0).wait()
        pltpu.sync_copy(rows1, acc.at[idx_d.at[j + 1]], add=True)
        return carry

    lax.fori_loop(0, nch_me // 2, body, 0)
    plsc.subcore_barrier()
    pltpu.sync_copy(acc.at[pl.ds(s * RPT, RPT)],
                    out.at[c].at[pl.ds(s * RPT, RPT)])


_R = 1000


def _gelu(v):
    return 0.5 * v * (1.0 + lax.erf(v * 0.7071067811865476))


def _tc_a(x, W1, degp):
    def body(x_ref, w_ref, deg_ref, h1s_ref, dinvb_ref):
        dsum = jnp.sum(deg_ref[...], axis=1, keepdims=True)
        dinv = lax.rsqrt(dsum + 1.0)
        dinvb = jnp.broadcast_to(dinv, (_R, H))
        h = jnp.dot(x_ref[...], w_ref[...],
                    preferred_element_type=jnp.float32)
        h1s_ref[...] = h * dinvb
        dinvb_ref[...] = dinvb

    return pl.pallas_call(
        body,
        grid=(N // _R,),
        in_specs=[
            pl.BlockSpec((_R, D), lambda m: (m, 0)),
            pl.BlockSpec((D, H), lambda m: (0, 0)),
            pl.BlockSpec((_R, NW), lambda m: (m, 0)),
        ],
        out_specs=[
            pl.BlockSpec((_R, H), lambda m: (m, 0)),
            pl.BlockSpec((_R, H), lambda m: (m, 0)),
        ],
        out_shape=[
            jax.ShapeDtypeStruct((N, H), jnp.float32),
            jax.ShapeDtypeStruct((N, H), jnp.float32),
        ],
    )(x, W1, degp)


def _tc_b(s1, h1s, dinvb, b1, W2):
    def body(s_ref, h1s_ref, dinvb_ref, b1_ref, w2_ref, h2s_ref):
        dinvb = dinvb_ref[...]
        agg = dinvb * (s_ref[0] + s_ref[1] + h1s_ref[...]) + b1_ref[...]
        y = _gelu(agg)
        mu = jnp.mean(y, axis=-1, keepdims=True)
        var = jnp.mean((y - mu) ** 2, axis=-1, keepdims=True)
        ln = (y - mu) / jnp.sqrt(var + 1e-5)
        h2 = jnp.dot(ln, w2_ref[...], preferred_element_type=jnp.float32)
        h2s_ref[...] = h2 * dinvb

    return pl.pallas_call(
        body,
        grid=(N // _R,),
        in_specs=[
            pl.BlockSpec((NC, _R, H), lambda m: (0, m, 0)),
            pl.BlockSpec((_R, H), lambda m: (m, 0)),
            pl.BlockSpec((_R, H), lambda m: (m, 0)),
            pl.BlockSpec((1, H), lambda m: (0, 0)),
            pl.BlockSpec((H, H), lambda m: (0, 0)),
        ],
        out_specs=pl.BlockSpec((_R, H), lambda m: (m, 0)),
        out_shape=jax.ShapeDtypeStruct((N, H), jnp.float32),
    )(s1, h1s, dinvb, b1, W2)


def _tc_c(s2, h2s, dinvb, b2):
    def body(s_ref, h2s_ref, dinvb_ref, b2_ref, out_ref):
        agg = (dinvb_ref[...] * (s_ref[0] + s_ref[1] + h2s_ref[...])
               + b2_ref[...])
        out_ref[...] = _gelu(agg)

    return pl.pallas_call(
        body,
        grid=(N // _R,),
        in_specs=[
            pl.BlockSpec((NC, _R, H), lambda m: (0, m, 0)),
            pl.BlockSpec((_R, H), lambda m: (m, 0)),
            pl.BlockSpec((_R, H), lambda m: (m, 0)),
            pl.BlockSpec((1, H), lambda m: (0, 0)),
        ],
        out_specs=pl.BlockSpec((_R, H), lambda m: (m, 0)),
        out_shape=jax.ShapeDtypeStruct((N, H), jnp.float32),
    )(s2, h2s, dinvb, b2)


def kernel(x, edge_index, W1, b1, W2, b2):
    src = edge_index[0]
    dst = edge_index[1]
    pad = EP - E
    srcp = jnp.concatenate(
        [src, jnp.zeros((pad,), jnp.int32)]).reshape(EP // CHUNK, CHUNK)
    dstp = jnp.concatenate(
        [dst, jnp.full((pad,), N, jnp.int32)]).reshape(EP // CHUNK, CHUNK)
    zeros = jnp.zeros((RPT, H), jnp.float32)

    degp = _sc_deg(dstp)
    h1s, dinvb = _tc_a(x, W1, degp[:, :N].T)
    s1 = _sc_pass(h1s, srcp, dstp, zeros)
    h2s = _tc_b(s1[:, :N, :], h1s, dinvb, b1.reshape(1, H), W2)
    s2 = _sc_pass(h2s, srcp, dstp, zeros)
    return _tc_c(s2[:, :N, :], h2s, dinvb, b2.reshape(1, H))
